# baseline (speedup 1.0000x reference)
import functools

import jax
import jax.numpy as jnp
from jax import lax
from jax.experimental import pallas as pl
from jax.experimental.pallas import tpu as pltpu
from jax.experimental.pallas import tpu_sc as plsc

NUM_TABLES = 26
VOCAB = 100000
DIM = 32
BATCH = 4096

NC = 2
NS = 16
L = 16
HALF = VOCAB // 2
NVEC = BATCH // L

NUM_CHUNKS = 2
TC_CHUNK = NUM_TABLES // NUM_CHUNKS

_MESH = plsc.VectorSubcoreMesh(
    core_axis_name="c", subcore_axis_name="s", num_cores=NC, num_subcores=NS
)


def _make_emb_kernel(t_chunk):
    @functools.partial(
        pl.kernel,
        out_type=jax.ShapeDtypeStruct((t_chunk, DIM, BATCH), jnp.float32),
        mesh=_MESH,
        scratch_types=[
            pltpu.VMEM((2, HALF), jnp.float32),
            pltpu.VMEM((2, BATCH), jnp.int32),
            pltpu.VMEM((2, BATCH), jnp.float32),
            pltpu.SemaphoreType.DMA,
            pltpu.SemaphoreType.DMA,
            pltpu.SemaphoreType.DMA,
            pltpu.SemaphoreType.DMA,
            pltpu.SemaphoreType.DMA,
        ],
        compiler_params=pltpu.CompilerParams(
            use_tc_tiling_on_sc=False, needs_layout_passes=False
        ),
    )
    def _emb_kernel(tab, idx, out, rowbuf, idxbuf, outbuf, ssem0, ssem1, isem,
                    osem0, osem1):
        d = lax.axis_index("s") * NC + lax.axis_index("c")
        ssems = (ssem0, ssem1)
        osems = (osem0, osem1)

        def stage_copy(k, h):
            return pltpu.make_async_copy(
                tab.at[k, d, pl.ds(h * HALF, HALF)], rowbuf.at[h], ssems[h]
            )

        def idx_copy(k, slot):
            return pltpu.make_async_copy(idx.at[k], idxbuf.at[slot], isem)

        def out_copy(k, slot):
            return pltpu.make_async_copy(
                outbuf.at[slot], out.at[k, d], osems[slot]
            )

        def gather_half(kslot, half, merge):
            base = half * HALF

            def body(i, c):
                sl = pl.ds(i * L, L)
                iv = idxbuf[kslot, sl]
                pos = iv - base
                if half == 0:
                    m = iv < HALF
                else:
                    m = iv >= HALF
                g = plsc.load_gather(rowbuf.at[half], [pos], mask=m)
                if merge:
                    outbuf[kslot, sl] = jnp.where(m, g, outbuf[kslot, sl])
                else:
                    outbuf[kslot, sl] = g
                return c

            lax.fori_loop(0, NVEC, body, 0, unroll=8)

        def process_table(k, kslot):
            stage_copy(k, 1).start()
            idx_copy(k, kslot).wait()

            @pl.when(k + 1 < t_chunk)
            def _():
                idx_copy(k + 1, 1 - kslot).start()

            @pl.when(k >= 2)
            def _():
                out_copy(k - 2, kslot).wait()

            stage_copy(k, 0).wait()
            gather_half(kslot, 0, merge=False)

            @pl.when(k + 1 < t_chunk)
            def _():
                stage_copy(k + 1, 0).start()

            stage_copy(k, 1).wait()
            gather_half(kslot, 1, merge=True)

            out_copy(k, kslot).start()

        idx_copy(0, 0).start()
        stage_copy(0, 0).start()

        def table_pair(j, c):
            process_table(2 * j, 0)
            process_table(2 * j + 1, 1)
            return c

        lax.fori_loop(0, t_chunk // 2, table_pair, 0)
        if t_chunk % 2:
            process_table(jnp.int32(t_chunk - 1), (t_chunk - 1) % 2)

        out_copy(t_chunk - 2, (t_chunk - 2) % 2).wait()
        out_copy(t_chunk - 1, (t_chunk - 1) % 2).wait()

    return _emb_kernel


_EMB_CHUNK = _make_emb_kernel(TC_CHUNK)


def kernel(tables, indices):
    idx_t = indices.T
    outs = []
    for g in range(NUM_CHUNKS):
        lo = g * TC_CHUNK
        t3 = tables[lo:lo + TC_CHUNK].transpose(0, 2, 1)
        outs.append(_EMB_CHUNK(t3, idx_t[lo:lo + TC_CHUNK]))
    out = jnp.concatenate(outs, axis=0)
    return out.transpose(2, 0, 1)


# speedup vs baseline: 1.2477x; 1.2477x over previous
import functools

import jax
import jax.numpy as jnp
from jax import lax
from jax.experimental import pallas as pl
from jax.experimental.pallas import tpu as pltpu
from jax.experimental.pallas import tpu_sc as plsc

NUM_TABLES = 26
VOCAB = 100000
DIM = 32
BATCH = 4096

NC = 2
NS = 16
L = 16
HALF = VOCAB // 2
NVEC = BATCH // L

_MESH = plsc.VectorSubcoreMesh(
    core_axis_name="c", subcore_axis_name="s", num_cores=NC, num_subcores=NS
)


def _make_emb_kernel(t_chunk):
    @functools.partial(
        pl.kernel,
        out_type=jax.ShapeDtypeStruct((t_chunk, DIM, BATCH), jnp.float32),
        mesh=_MESH,
        scratch_types=[
            pltpu.VMEM((2, HALF), jnp.float32),
            pltpu.VMEM((2, BATCH), jnp.int32),
            pltpu.VMEM((2, BATCH), jnp.float32),
            pltpu.SemaphoreType.DMA,
            pltpu.SemaphoreType.DMA,
            pltpu.SemaphoreType.DMA,
            pltpu.SemaphoreType.DMA,
            pltpu.SemaphoreType.DMA,
        ],
        compiler_params=pltpu.CompilerParams(
            use_tc_tiling_on_sc=False, needs_layout_passes=False
        ),
    )
    def _emb_kernel(tab, idx, out, rowbuf, idxbuf, outbuf, ssem0, ssem1, isem,
                    osem0, osem1):
        d = lax.axis_index("s") * NC + lax.axis_index("c")
        ssems = (ssem0, ssem1)
        osems = (osem0, osem1)

        def stage_copy(k, h):
            return pltpu.make_async_copy(
                tab.at[k, d, pl.ds(h * HALF, HALF)], rowbuf.at[h], ssems[h]
            )

        def idx_copy(k, slot):
            return pltpu.make_async_copy(idx.at[k], idxbuf.at[slot], isem)

        def out_copy(k, slot):
            return pltpu.make_async_copy(
                outbuf.at[slot], out.at[k, d], osems[slot]
            )

        def gather_half(kslot, half, merge):
            base = half * HALF

            def body(i, c):
                sl = pl.ds(i * L, L)
                iv = idxbuf[kslot, sl]
                pos = iv - base
                if half == 0:
                    m = iv < HALF
                else:
                    m = iv >= HALF
                g = plsc.load_gather(rowbuf.at[half], [pos], mask=m)
                if merge:
                    outbuf[kslot, sl] = jnp.where(m, g, outbuf[kslot, sl])
                else:
                    outbuf[kslot, sl] = g
                return c

            lax.fori_loop(0, NVEC, body, 0, unroll=8)

        def process_table(k, kslot):
            stage_copy(k, 1).start()
            idx_copy(k, kslot).wait()

            @pl.when(k + 1 < t_chunk)
            def _():
                idx_copy(k + 1, 1 - kslot).start()

            @pl.when(k >= 2)
            def _():
                out_copy(k - 2, kslot).wait()

            stage_copy(k, 0).wait()
            gather_half(kslot, 0, merge=False)

            @pl.when(k + 1 < t_chunk)
            def _():
                stage_copy(k + 1, 0).start()

            stage_copy(k, 1).wait()
            gather_half(kslot, 1, merge=True)

            out_copy(k, kslot).start()

        idx_copy(0, 0).start()
        stage_copy(0, 0).start()

        def table_pair(j, c):
            process_table(2 * j, 0)
            process_table(2 * j + 1, 1)
            return c

        lax.fori_loop(0, t_chunk // 2, table_pair, 0)
        if t_chunk % 2:
            process_table(jnp.int32(t_chunk - 1), (t_chunk - 1) % 2)

        out_copy(t_chunk - 2, (t_chunk - 2) % 2).wait()
        out_copy(t_chunk - 1, (t_chunk - 1) % 2).wait()

    return _emb_kernel


_EMB_KERNEL = _make_emb_kernel(NUM_TABLES)


def kernel(tables, indices):
    t3 = tables.transpose(0, 2, 1)
    idx_t = indices.T
    out = _EMB_KERNEL(t3, idx_t)
    return out.transpose(2, 0, 1)
